# baseline (speedup 1.0000x reference)
import jax
import jax.numpy as jnp
from jax import lax
from jax.experimental import pallas as pl
from jax.experimental.pallas import tpu as pltpu
from jax.experimental.pallas import tpu_sc as plsc

_N = 100000
_B = 256
_ANN = 128
_H = 64

_BN = 8192
_GRID = (_N + _BN - 1) // _BN

_NW = 32
_CH = 3136
_LAST = _N - (_NW - 1) * _CH


def _stage1(ann_ref, wred_ref, wg_ref, wo_ref, s_ref, t_ref):
    wg2 = wg_ref[0:_H, :] + wg_ref[_H:2 * _H, :]
    wo2 = wo_ref[0:_H, :] + wo_ref[_H:2 * _H, :]
    w2 = jnp.dot(wred_ref[...], jnp.concatenate([wg2, wo2], axis=1),
                 preferred_element_type=jnp.float32)
    st = jnp.dot(ann_ref[...], w2, preferred_element_type=jnp.float32)
    stt = jnp.transpose(st)
    s_ref[...] = stt[0]
    t_ref[...] = stt[1]


def _take16(x, idx):
    dnums = lax.GatherDimensionNumbers(
        offset_dims=(), collapsed_slice_dims=(0,), start_index_map=(0,))
    return lax.gather(x, idx[:, None], dnums, (1,),
                      mode=lax.GatherScatterMode.PROMISE_IN_BOUNDS)


def _stage2_call(s1, t1, segi):
    mesh = plsc.VectorSubcoreMesh(core_axis_name="c", subcore_axis_name="s")
    part = jax.ShapeDtypeStruct((_NW, _B), jnp.float32)

    @pl.kernel(
        out_type=[part, part, part],
        mesh=mesh,
        compiler_params=pltpu.CompilerParams(needs_layout_passes=False),
        scratch_types=[
            pltpu.VMEM((_CH,), jnp.float32),
            pltpu.VMEM((_CH,), jnp.float32),
            pltpu.VMEM((_CH,), jnp.int32),
            pltpu.VMEM((_B,), jnp.float32),
            pltpu.VMEM((_B,), jnp.float32),
            pltpu.VMEM((_B,), jnp.float32),
        ],
    )
    def sc_kernel(s_hbm, t_hbm, id_hbm, mo_hbm, do_hbm, to_hbm,
                  s_loc, t_loc, id_loc, m_tab, d_tab, t_tab):
        wid = lax.axis_index("s") * 2 + lax.axis_index("c")
        base = wid * _CH
        nv = jnp.where(wid == _NW - 1, _LAST // 16, _CH // 16)

        pltpu.sync_copy(s_hbm.at[pl.ds(base, _LAST)], s_loc.at[pl.ds(0, _LAST)])
        pltpu.sync_copy(t_hbm.at[pl.ds(base, _LAST)], t_loc.at[pl.ds(0, _LAST)])
        pltpu.sync_copy(id_hbm.at[pl.ds(base, _LAST)], id_loc.at[pl.ds(0, _LAST)])

        @pl.when(wid < _NW - 1)
        def _tail():
            rest = _CH - _LAST
            pltpu.sync_copy(s_hbm.at[pl.ds(base + _LAST, rest)],
                            s_loc.at[pl.ds(_LAST, rest)])
            pltpu.sync_copy(t_hbm.at[pl.ds(base + _LAST, rest)],
                            t_loc.at[pl.ds(_LAST, rest)])
            pltpu.sync_copy(id_hbm.at[pl.ds(base + _LAST, rest)],
                            id_loc.at[pl.ds(_LAST, rest)])

        minus_inf = jnp.full((16,), -jnp.inf, jnp.float32)
        zeros16 = jnp.zeros((16,), jnp.float32)

        @pl.loop(0, _B, step=16)
        def _init(o):
            oo = pl.multiple_of(o, 16)
            m_tab[pl.ds(oo, 16)] = minus_inf
            d_tab[pl.ds(oo, 16)] = zeros16
            t_tab[pl.ds(oo, 16)] = zeros16

        iota16 = lax.broadcasted_iota(jnp.int32, (16,), 0)
        shift_idx = [jnp.maximum(iota16 - k, 0) for k in (1, 2, 4, 8)]
        next_idx = jnp.minimum(iota16 + 1, 15)

        def _run_masks(ids):
            oks = []
            for k, idx in zip((1, 2, 4, 8), shift_idx):
                oks.append((_take16(ids, idx) == ids) & (iota16 >= k))
            last = (ids != _take16(ids, next_idx)) | (iota16 == 15)
            return oks, last

        def _sweep_max(j, _):
            off = pl.multiple_of(j * 16, 16)
            ids = id_loc[pl.ds(off, 16)]
            acc = s_loc[pl.ds(off, 16)]
            oks, last = _run_masks(ids)
            for ok, idx in zip(oks, shift_idx):
                acc = jnp.maximum(
                    acc, jnp.where(ok, _take16(acc, idx), -jnp.inf))
            cur = plsc.load_gather(m_tab, [ids])
            plsc.store_scatter(m_tab, [ids], jnp.maximum(cur, acc), mask=last)
            return _

        lax.fori_loop(0, nv, _sweep_max, None)

        def _sweep_sum(j, _):
            off = pl.multiple_of(j * 16, 16)
            ids = id_loc[pl.ds(off, 16)]
            s16 = s_loc[pl.ds(off, 16)]
            t16 = t_loc[pl.ds(off, 16)]
            m16 = plsc.load_gather(m_tab, [ids])
            e = jnp.exp(s16 - m16)
            et = e * t16
            oks, last = _run_masks(ids)
            for ok, idx in zip(oks, shift_idx):
                e = e + jnp.where(ok, _take16(e, idx), 0.0)
                et = et + jnp.where(ok, _take16(et, idx), 0.0)
            plsc.addupdate_scatter(d_tab, [ids], e, mask=last)
            plsc.addupdate_scatter(t_tab, [ids], et, mask=last)
            return _

        lax.fori_loop(0, nv, _sweep_sum, None)

        pltpu.sync_copy(m_tab, mo_hbm.at[wid])
        pltpu.sync_copy(d_tab, do_hbm.at[wid])
        pltpu.sync_copy(t_tab, to_hbm.at[wid])

    return sc_kernel(s1, t1, segi)


def _stage3(mo_ref, do_ref, to_ref, bred_ref, wo_ref, bo_ref, out_ref):
    mw = mo_ref[...]
    m = jnp.max(mw, axis=0, keepdims=True)
    scale = jnp.where(mw == -jnp.inf, 0.0, jnp.exp(mw - m))
    d = jnp.sum(do_ref[...] * scale, axis=0, keepdims=True)
    t = jnp.sum(to_ref[...] * scale, axis=0, keepdims=True)
    wo2 = wo_ref[0:_H, :] + wo_ref[_H:2 * _H, :]
    c1 = jnp.dot(bred_ref[...], wo2, preferred_element_type=jnp.float32)
    out_ref[...] = (t / jnp.maximum(d, 1e-12)
                    + jnp.where(d > 0, 1.0, 0.0) * c1[0, 0]
                    + bo_ref[0, 0])


def kernel(annotation, segment_ids, W_reduce, b_reduce, W_gate, b_gate,
           W_out, b_out):
    segi = segment_ids.astype(jnp.int32)

    s1, t1 = pl.pallas_call(
        _stage1,
        grid=(_GRID,),
        in_specs=[
            pl.BlockSpec((_BN, _ANN), lambda i: (i, 0)),
            pl.BlockSpec((_ANN, _H), lambda i: (0, 0)),
            pl.BlockSpec((2 * _H, 1), lambda i: (0, 0)),
            pl.BlockSpec((2 * _H, 1), lambda i: (0, 0)),
        ],
        out_specs=[
            pl.BlockSpec((_BN,), lambda i: (i,)),
            pl.BlockSpec((_BN,), lambda i: (i,)),
        ],
        out_shape=[
            jax.ShapeDtypeStruct((_N,), jnp.float32),
            jax.ShapeDtypeStruct((_N,), jnp.float32),
        ],
    )(annotation, W_reduce, W_gate, W_out)

    mo, do, to = _stage2_call(s1, t1, segi)

    out = pl.pallas_call(
        _stage3,
        in_specs=[
            pl.BlockSpec((_NW, _B), lambda: (0, 0)),
            pl.BlockSpec((_NW, _B), lambda: (0, 0)),
            pl.BlockSpec((_NW, _B), lambda: (0, 0)),
            pl.BlockSpec((1, _H), lambda: (0, 0)),
            pl.BlockSpec((2 * _H, 1), lambda: (0, 0)),
            pl.BlockSpec((1, 1), lambda: (0, 0)),
        ],
        out_specs=pl.BlockSpec((1, _B), lambda: (0, 0)),
        out_shape=jax.ShapeDtypeStruct((1, _B), jnp.float32),
    )(mo, do, to, b_reduce.reshape(1, _H), W_out, b_out.reshape(1, 1))
    return out.reshape(_B)


# speedup vs baseline: 1.0376x; 1.0376x over previous
import jax
import jax.numpy as jnp
from jax import lax
from jax.experimental import pallas as pl
from jax.experimental.pallas import tpu as pltpu
from jax.experimental.pallas import tpu_sc as plsc

_N = 100000
_B = 256
_ANN = 128
_H = 64

_BN = 16384
_GRID = (_N + _BN - 1) // _BN

_NW = 32
_CH = 3136
_LAST = _N - (_NW - 1) * _CH


def _stage1(ann_ref, wred_ref, wg_ref, wo_ref, s_ref, t_ref):
    wg2 = wg_ref[0:_H, :] + wg_ref[_H:2 * _H, :]
    wo2 = wo_ref[0:_H, :] + wo_ref[_H:2 * _H, :]
    w2 = jnp.dot(wred_ref[...], jnp.concatenate([wg2, wo2], axis=1),
                 preferred_element_type=jnp.float32)
    st = jnp.dot(ann_ref[...], w2, preferred_element_type=jnp.float32)
    stt = jnp.transpose(st)
    s_ref[...] = stt[0]
    t_ref[...] = stt[1]


def _take16(x, idx):
    dnums = lax.GatherDimensionNumbers(
        offset_dims=(), collapsed_slice_dims=(0,), start_index_map=(0,))
    return lax.gather(x, idx[:, None], dnums, (1,),
                      mode=lax.GatherScatterMode.PROMISE_IN_BOUNDS)


def _stage2_call(s1, t1, segi):
    mesh = plsc.VectorSubcoreMesh(core_axis_name="c", subcore_axis_name="s")
    part = jax.ShapeDtypeStruct((_NW, _B), jnp.float32)

    @pl.kernel(
        out_type=[part, part, part],
        mesh=mesh,
        compiler_params=pltpu.CompilerParams(needs_layout_passes=False),
        scratch_types=[
            pltpu.VMEM((_CH,), jnp.float32),
            pltpu.VMEM((_CH,), jnp.float32),
            pltpu.VMEM((_CH,), jnp.int32),
            pltpu.VMEM((_B,), jnp.float32),
            pltpu.VMEM((_B,), jnp.float32),
            pltpu.VMEM((_B,), jnp.float32),
        ],
    )
    def sc_kernel(s_hbm, t_hbm, id_hbm, mo_hbm, do_hbm, to_hbm,
                  s_loc, t_loc, id_loc, m_tab, d_tab, t_tab):
        wid = lax.axis_index("s") * 2 + lax.axis_index("c")
        base = wid * _CH
        nv = jnp.where(wid == _NW - 1, _LAST // 16, _CH // 16)

        pltpu.sync_copy(s_hbm.at[pl.ds(base, _LAST)], s_loc.at[pl.ds(0, _LAST)])
        pltpu.sync_copy(t_hbm.at[pl.ds(base, _LAST)], t_loc.at[pl.ds(0, _LAST)])
        pltpu.sync_copy(id_hbm.at[pl.ds(base, _LAST)], id_loc.at[pl.ds(0, _LAST)])

        @pl.when(wid < _NW - 1)
        def _tail():
            rest = _CH - _LAST
            pltpu.sync_copy(s_hbm.at[pl.ds(base + _LAST, rest)],
                            s_loc.at[pl.ds(_LAST, rest)])
            pltpu.sync_copy(t_hbm.at[pl.ds(base + _LAST, rest)],
                            t_loc.at[pl.ds(_LAST, rest)])
            pltpu.sync_copy(id_hbm.at[pl.ds(base + _LAST, rest)],
                            id_loc.at[pl.ds(_LAST, rest)])

        minus_inf = jnp.full((16,), -jnp.inf, jnp.float32)
        zeros16 = jnp.zeros((16,), jnp.float32)

        @pl.loop(0, _B, step=16)
        def _init(o):
            oo = pl.multiple_of(o, 16)
            m_tab[pl.ds(oo, 16)] = minus_inf
            d_tab[pl.ds(oo, 16)] = zeros16
            t_tab[pl.ds(oo, 16)] = zeros16

        iota16 = lax.broadcasted_iota(jnp.int32, (16,), 0)
        shift_idx = [jnp.maximum(iota16 - k, 0) for k in (1, 2, 4, 8)]
        next_idx = jnp.minimum(iota16 + 1, 15)

        def _run_masks(ids):
            oks = []
            for k, idx in zip((1, 2, 4, 8), shift_idx):
                oks.append((_take16(ids, idx) == ids) & (iota16 >= k))
            last = (ids != _take16(ids, next_idx)) | (iota16 == 15)
            return oks, last

        def _sweep_max(j, _):
            off = pl.multiple_of(j * 16, 16)
            ids = id_loc[pl.ds(off, 16)]
            acc = s_loc[pl.ds(off, 16)]
            oks, last = _run_masks(ids)
            for ok, idx in zip(oks, shift_idx):
                acc = jnp.maximum(
                    acc, jnp.where(ok, _take16(acc, idx), -jnp.inf))
            cur = plsc.load_gather(m_tab, [ids])
            plsc.store_scatter(m_tab, [ids], jnp.maximum(cur, acc), mask=last)
            return _

        lax.fori_loop(0, nv, _sweep_max, None)

        def _sweep_sum(j, _):
            off = pl.multiple_of(j * 16, 16)
            ids = id_loc[pl.ds(off, 16)]
            s16 = s_loc[pl.ds(off, 16)]
            t16 = t_loc[pl.ds(off, 16)]
            m16 = plsc.load_gather(m_tab, [ids])
            e = jnp.exp(s16 - m16)
            et = e * t16
            oks, last = _run_masks(ids)
            for ok, idx in zip(oks, shift_idx):
                e = e + jnp.where(ok, _take16(e, idx), 0.0)
                et = et + jnp.where(ok, _take16(et, idx), 0.0)
            plsc.addupdate_scatter(d_tab, [ids], e, mask=last)
            plsc.addupdate_scatter(t_tab, [ids], et, mask=last)
            return _

        lax.fori_loop(0, nv, _sweep_sum, None)

        pltpu.sync_copy(m_tab, mo_hbm.at[wid])
        pltpu.sync_copy(d_tab, do_hbm.at[wid])
        pltpu.sync_copy(t_tab, to_hbm.at[wid])

    return sc_kernel(s1, t1, segi)


def _stage3(mo_ref, do_ref, to_ref, bred_ref, wo_ref, bo_ref, out_ref):
    mw = mo_ref[...]
    m = jnp.max(mw, axis=0, keepdims=True)
    scale = jnp.where(mw == -jnp.inf, 0.0, jnp.exp(mw - m))
    d = jnp.sum(do_ref[...] * scale, axis=0, keepdims=True)
    t = jnp.sum(to_ref[...] * scale, axis=0, keepdims=True)
    wo2 = wo_ref[0:_H, :] + wo_ref[_H:2 * _H, :]
    c1 = jnp.dot(bred_ref[...], wo2, preferred_element_type=jnp.float32)
    out_ref[...] = (t / jnp.maximum(d, 1e-12)
                    + jnp.where(d > 0, 1.0, 0.0) * c1[0, 0]
                    + bo_ref[0, 0])


def kernel(annotation, segment_ids, W_reduce, b_reduce, W_gate, b_gate,
           W_out, b_out):
    segi = segment_ids.astype(jnp.int32)

    s1, t1 = pl.pallas_call(
        _stage1,
        grid=(_GRID,),
        in_specs=[
            pl.BlockSpec((_BN, _ANN), lambda i: (i, 0)),
            pl.BlockSpec((_ANN, _H), lambda i: (0, 0)),
            pl.BlockSpec((2 * _H, 1), lambda i: (0, 0)),
            pl.BlockSpec((2 * _H, 1), lambda i: (0, 0)),
        ],
        out_specs=[
            pl.BlockSpec((_BN,), lambda i: (i,)),
            pl.BlockSpec((_BN,), lambda i: (i,)),
        ],
        out_shape=[
            jax.ShapeDtypeStruct((_N,), jnp.float32),
            jax.ShapeDtypeStruct((_N,), jnp.float32),
        ],
    )(annotation, W_reduce, W_gate, W_out)

    mo, do, to = _stage2_call(s1, t1, segi)

    out = pl.pallas_call(
        _stage3,
        in_specs=[
            pl.BlockSpec((_NW, _B), lambda: (0, 0)),
            pl.BlockSpec((_NW, _B), lambda: (0, 0)),
            pl.BlockSpec((_NW, _B), lambda: (0, 0)),
            pl.BlockSpec((1, _H), lambda: (0, 0)),
            pl.BlockSpec((2 * _H, 1), lambda: (0, 0)),
            pl.BlockSpec((1, 1), lambda: (0, 0)),
        ],
        out_specs=pl.BlockSpec((1, _B), lambda: (0, 0)),
        out_shape=jax.ShapeDtypeStruct((1, _B), jnp.float32),
    )(mo, do, to, b_reduce.reshape(1, _H), W_out, b_out.reshape(1, 1))
    return out.reshape(_B)


# speedup vs baseline: 1.1026x; 1.0627x over previous
import jax
import jax.numpy as jnp
from jax import lax
from jax.experimental import pallas as pl
from jax.experimental.pallas import tpu as pltpu
from jax.experimental.pallas import tpu_sc as plsc

_N = 100000
_B = 256
_ANN = 128
_H = 64

_BN = 16384
_GRID = (_N + _BN - 1) // _BN

_NW = 32
_CH = 3136
_LAST = _N - (_NW - 1) * _CH


def _stage1(ann_ref, wred_ref, wg_ref, wo_ref, s_ref, t_ref):
    wg2 = wg_ref[0:_H, :] + wg_ref[_H:2 * _H, :]
    wo2 = wo_ref[0:_H, :] + wo_ref[_H:2 * _H, :]
    w2 = jnp.dot(wred_ref[...], jnp.concatenate([wg2, wo2], axis=1),
                 preferred_element_type=jnp.float32)
    st = jnp.dot(ann_ref[...], w2, preferred_element_type=jnp.float32)
    stt = jnp.transpose(st)
    s_ref[...] = stt[0]
    t_ref[...] = stt[1]


def _take16(x, idx):
    dnums = lax.GatherDimensionNumbers(
        offset_dims=(), collapsed_slice_dims=(0,), start_index_map=(0,))
    return lax.gather(x, idx[:, None], dnums, (1,),
                      mode=lax.GatherScatterMode.PROMISE_IN_BOUNDS)


def _stage2_call(s1, t1, segi):
    mesh = plsc.VectorSubcoreMesh(core_axis_name="c", subcore_axis_name="s")
    part = jax.ShapeDtypeStruct((_NW, _B), jnp.float32)

    @pl.kernel(
        out_type=[part, part, part],
        mesh=mesh,
        compiler_params=pltpu.CompilerParams(needs_layout_passes=False),
        scratch_types=[
            pltpu.VMEM((_CH,), jnp.float32),
            pltpu.VMEM((_CH,), jnp.float32),
            pltpu.VMEM((_CH,), jnp.int32),
            pltpu.VMEM((_B,), jnp.float32),
            pltpu.VMEM((_B,), jnp.float32),
            pltpu.VMEM((_B,), jnp.float32),
        ],
    )
    def sc_kernel(s_hbm, t_hbm, id_hbm, mo_hbm, do_hbm, to_hbm,
                  s_loc, t_loc, id_loc, m_tab, d_tab, t_tab):
        wid = lax.axis_index("s") * 2 + lax.axis_index("c")
        base = wid * _CH
        nv = jnp.where(wid == _NW - 1, _LAST // 16, _CH // 16)

        pltpu.sync_copy(s_hbm.at[pl.ds(base, _LAST)], s_loc.at[pl.ds(0, _LAST)])
        pltpu.sync_copy(t_hbm.at[pl.ds(base, _LAST)], t_loc.at[pl.ds(0, _LAST)])
        pltpu.sync_copy(id_hbm.at[pl.ds(base, _LAST)], id_loc.at[pl.ds(0, _LAST)])

        @pl.when(wid < _NW - 1)
        def _tail():
            rest = _CH - _LAST
            pltpu.sync_copy(s_hbm.at[pl.ds(base + _LAST, rest)],
                            s_loc.at[pl.ds(_LAST, rest)])
            pltpu.sync_copy(t_hbm.at[pl.ds(base + _LAST, rest)],
                            t_loc.at[pl.ds(_LAST, rest)])
            pltpu.sync_copy(id_hbm.at[pl.ds(base + _LAST, rest)],
                            id_loc.at[pl.ds(_LAST, rest)])

        minus_inf = jnp.full((16,), -jnp.inf, jnp.float32)
        zeros16 = jnp.zeros((16,), jnp.float32)

        @pl.loop(0, _B, step=16)
        def _init(o):
            oo = pl.multiple_of(o, 16)
            m_tab[pl.ds(oo, 16)] = minus_inf
            d_tab[pl.ds(oo, 16)] = zeros16
            t_tab[pl.ds(oo, 16)] = zeros16

        iota16 = lax.broadcasted_iota(jnp.int32, (16,), 0)
        shift_idx = [jnp.maximum(iota16 - k, 0) for k in (1, 2, 4, 8)]
        next_idx = jnp.minimum(iota16 + 1, 15)

        def _run_masks(ids):
            oks = []
            for k, idx in zip((1, 2, 4, 8), shift_idx):
                oks.append((_take16(ids, idx) == ids) & (iota16 >= k))
            last = (ids != _take16(ids, next_idx)) | (iota16 == 15)
            return oks, last

        def _seg_max(off):
            ids = id_loc[pl.ds(off, 16)]
            acc = s_loc[pl.ds(off, 16)]
            for idx in shift_idx:
                ok = _take16(ids, idx) == ids
                acc = jnp.maximum(
                    acc, jnp.where(ok, _take16(acc, idx), -jnp.inf))
            last = (ids != _take16(ids, next_idx)) | (iota16 == 15)
            return ids, acc, last

        def _sweep_max(j, _):
            off = pl.multiple_of(j * 32, 16)
            ids_a, acc_a, last_a = _seg_max(off)
            ids_b, acc_b, last_b = _seg_max(off + 16)
            cur_a = plsc.load_gather(m_tab, [ids_a])
            plsc.store_scatter(m_tab, [ids_a], jnp.maximum(cur_a, acc_a),
                               mask=last_a)
            cur_b = plsc.load_gather(m_tab, [ids_b])
            plsc.store_scatter(m_tab, [ids_b], jnp.maximum(cur_b, acc_b),
                               mask=last_b)
            return _

        lax.fori_loop(0, nv // 2, _sweep_max, None)

        def _seg_sums(off):
            ids = id_loc[pl.ds(off, 16)]
            s16 = s_loc[pl.ds(off, 16)]
            t16 = t_loc[pl.ds(off, 16)]
            m16 = plsc.load_gather(m_tab, [ids])
            e = jnp.exp(s16 - m16)
            et = e * t16
            oks, last = _run_masks(ids)
            for ok, idx in zip(oks, shift_idx):
                e = e + jnp.where(ok, _take16(e, idx), 0.0)
                et = et + jnp.where(ok, _take16(et, idx), 0.0)
            return ids, e, et, last

        def _sweep_sum(j, _):
            off = pl.multiple_of(j * 32, 16)
            ids_a, e_a, et_a, last_a = _seg_sums(off)
            ids_b, e_b, et_b, last_b = _seg_sums(off + 16)
            plsc.addupdate_scatter(d_tab, [ids_a], e_a, mask=last_a)
            plsc.addupdate_scatter(t_tab, [ids_a], et_a, mask=last_a)
            plsc.addupdate_scatter(d_tab, [ids_b], e_b, mask=last_b)
            plsc.addupdate_scatter(t_tab, [ids_b], et_b, mask=last_b)
            return _

        lax.fori_loop(0, nv // 2, _sweep_sum, None)

        pltpu.sync_copy(m_tab, mo_hbm.at[wid])
        pltpu.sync_copy(d_tab, do_hbm.at[wid])
        pltpu.sync_copy(t_tab, to_hbm.at[wid])

    return sc_kernel(s1, t1, segi)


def _stage3(mo_ref, do_ref, to_ref, bred_ref, wo_ref, bo_ref, out_ref):
    mw = mo_ref[...]
    m = jnp.max(mw, axis=0, keepdims=True)
    scale = jnp.where(mw == -jnp.inf, 0.0, jnp.exp(mw - m))
    d = jnp.sum(do_ref[...] * scale, axis=0, keepdims=True)
    t = jnp.sum(to_ref[...] * scale, axis=0, keepdims=True)
    wo2 = wo_ref[0:_H, :] + wo_ref[_H:2 * _H, :]
    c1 = jnp.dot(bred_ref[...], wo2, preferred_element_type=jnp.float32)
    out_ref[...] = (t / jnp.maximum(d, 1e-12)
                    + jnp.where(d > 0, 1.0, 0.0) * c1[0, 0]
                    + bo_ref[0, 0])


def kernel(annotation, segment_ids, W_reduce, b_reduce, W_gate, b_gate,
           W_out, b_out):
    segi = segment_ids.astype(jnp.int32)

    s1, t1 = pl.pallas_call(
        _stage1,
        grid=(_GRID,),
        in_specs=[
            pl.BlockSpec((_BN, _ANN), lambda i: (i, 0)),
            pl.BlockSpec((_ANN, _H), lambda i: (0, 0)),
            pl.BlockSpec((2 * _H, 1), lambda i: (0, 0)),
            pl.BlockSpec((2 * _H, 1), lambda i: (0, 0)),
        ],
        out_specs=[
            pl.BlockSpec((_BN,), lambda i: (i,)),
            pl.BlockSpec((_BN,), lambda i: (i,)),
        ],
        out_shape=[
            jax.ShapeDtypeStruct((_N,), jnp.float32),
            jax.ShapeDtypeStruct((_N,), jnp.float32),
        ],
    )(annotation, W_reduce, W_gate, W_out)

    mo, do, to = _stage2_call(s1, t1, segi)

    out = pl.pallas_call(
        _stage3,
        in_specs=[
            pl.BlockSpec((_NW, _B), lambda: (0, 0)),
            pl.BlockSpec((_NW, _B), lambda: (0, 0)),
            pl.BlockSpec((_NW, _B), lambda: (0, 0)),
            pl.BlockSpec((1, _H), lambda: (0, 0)),
            pl.BlockSpec((2 * _H, 1), lambda: (0, 0)),
            pl.BlockSpec((1, 1), lambda: (0, 0)),
        ],
        out_specs=pl.BlockSpec((1, _B), lambda: (0, 0)),
        out_shape=jax.ShapeDtypeStruct((1, _B), jnp.float32),
    )(mo, do, to, b_reduce.reshape(1, _H), W_out, b_out.reshape(1, 1))
    return out.reshape(_B)


# speedup vs baseline: 1.1575x; 1.0498x over previous
import jax
import jax.numpy as jnp
from jax import lax
from jax.experimental import pallas as pl
from jax.experimental.pallas import tpu as pltpu
from jax.experimental.pallas import tpu_sc as plsc

_N = 100000
_B = 256
_ANN = 128
_H = 64

_BN = 16384
_GRID = (_N + _BN - 1) // _BN

_NW = 32
_CH = 3136
_LAST = _N - (_NW - 1) * _CH


def _stage1(ann_ref, wred_ref, wg_ref, wo_ref, s_ref, t_ref):
    wg2 = wg_ref[0:_H, :] + wg_ref[_H:2 * _H, :]
    wo2 = wo_ref[0:_H, :] + wo_ref[_H:2 * _H, :]
    w2 = jnp.dot(wred_ref[...], jnp.concatenate([wg2, wo2], axis=1),
                 preferred_element_type=jnp.float32)
    st = jnp.dot(ann_ref[...], w2, preferred_element_type=jnp.float32)
    stt = jnp.transpose(st)
    s_ref[...] = stt[0]
    t_ref[...] = stt[1]


def _take16(x, idx):
    dnums = lax.GatherDimensionNumbers(
        offset_dims=(), collapsed_slice_dims=(0,), start_index_map=(0,))
    return lax.gather(x, idx[:, None], dnums, (1,),
                      mode=lax.GatherScatterMode.PROMISE_IN_BOUNDS)


def _stage2_call(s1, t1, segi):
    mesh = plsc.VectorSubcoreMesh(core_axis_name="c", subcore_axis_name="s")
    part = jax.ShapeDtypeStruct((_NW, _B), jnp.float32)

    @pl.kernel(
        out_type=[part, part, part],
        mesh=mesh,
        compiler_params=pltpu.CompilerParams(needs_layout_passes=False),
        scratch_types=[
            pltpu.VMEM((_CH,), jnp.float32),
            pltpu.VMEM((_CH,), jnp.float32),
            pltpu.VMEM((_CH,), jnp.int32),
            pltpu.VMEM((_B,), jnp.float32),
            pltpu.VMEM((_B,), jnp.float32),
            pltpu.VMEM((_B,), jnp.float32),
            pltpu.SemaphoreType.DMA,
        ],
    )
    def sc_kernel(s_hbm, t_hbm, id_hbm, mo_hbm, do_hbm, to_hbm,
                  s_loc, t_loc, id_loc, m_tab, d_tab, t_tab, sem):
        wid = lax.axis_index("s") * 2 + lax.axis_index("c")
        base = wid * _CH
        nv = jnp.where(wid == _NW - 1, _LAST // 16, _CH // 16)

        rest = _CH - _LAST
        cps = [
            pltpu.make_async_copy(s_hbm.at[pl.ds(base, _LAST)],
                                  s_loc.at[pl.ds(0, _LAST)], sem),
            pltpu.make_async_copy(t_hbm.at[pl.ds(base, _LAST)],
                                  t_loc.at[pl.ds(0, _LAST)], sem),
            pltpu.make_async_copy(id_hbm.at[pl.ds(base, _LAST)],
                                  id_loc.at[pl.ds(0, _LAST)], sem),
        ]
        tails = [
            pltpu.make_async_copy(s_hbm.at[pl.ds(base + _LAST, rest)],
                                  s_loc.at[pl.ds(_LAST, rest)], sem),
            pltpu.make_async_copy(t_hbm.at[pl.ds(base + _LAST, rest)],
                                  t_loc.at[pl.ds(_LAST, rest)], sem),
            pltpu.make_async_copy(id_hbm.at[pl.ds(base + _LAST, rest)],
                                  id_loc.at[pl.ds(_LAST, rest)], sem),
        ]
        for cp in cps:
            cp.start()

        @pl.when(wid < _NW - 1)
        def _tail_start():
            for cp in tails:
                cp.start()

        for cp in cps:
            cp.wait()

        @pl.when(wid < _NW - 1)
        def _tail_wait():
            for cp in tails:
                cp.wait()

        minus_inf = jnp.full((16,), -jnp.inf, jnp.float32)
        zeros16 = jnp.zeros((16,), jnp.float32)

        @pl.loop(0, _B, step=16)
        def _init(o):
            oo = pl.multiple_of(o, 16)
            m_tab[pl.ds(oo, 16)] = minus_inf
            d_tab[pl.ds(oo, 16)] = zeros16
            t_tab[pl.ds(oo, 16)] = zeros16

        iota16 = lax.broadcasted_iota(jnp.int32, (16,), 0)
        shift_idx = [jnp.maximum(iota16 - k, 0) for k in (1, 2, 4, 8)]
        next_idx = jnp.minimum(iota16 + 1, 15)

        def _run_masks(ids):
            oks = []
            for k, idx in zip((1, 2, 4, 8), shift_idx):
                oks.append((_take16(ids, idx) == ids) & (iota16 >= k))
            last = (ids != _take16(ids, next_idx)) | (iota16 == 15)
            return oks, last

        def _seg_max(off):
            ids = id_loc[pl.ds(off, 16)]
            acc = s_loc[pl.ds(off, 16)]
            for idx in shift_idx:
                ok = _take16(ids, idx) == ids
                acc = jnp.maximum(
                    acc, jnp.where(ok, _take16(acc, idx), -jnp.inf))
            last = (ids != _take16(ids, next_idx)) | (iota16 == 15)
            return ids, acc, last

        def _sweep_max(j, _):
            off = pl.multiple_of(j * 32, 16)
            ids_a, acc_a, last_a = _seg_max(off)
            ids_b, acc_b, last_b = _seg_max(off + 16)
            cur_a = plsc.load_gather(m_tab, [ids_a])
            plsc.store_scatter(m_tab, [ids_a], jnp.maximum(cur_a, acc_a),
                               mask=last_a)
            cur_b = plsc.load_gather(m_tab, [ids_b])
            plsc.store_scatter(m_tab, [ids_b], jnp.maximum(cur_b, acc_b),
                               mask=last_b)
            return _

        lax.fori_loop(0, nv // 2, _sweep_max, None)

        def _seg_sums(off):
            ids = id_loc[pl.ds(off, 16)]
            s16 = s_loc[pl.ds(off, 16)]
            t16 = t_loc[pl.ds(off, 16)]
            m16 = plsc.load_gather(m_tab, [ids])
            e = jnp.exp(s16 - m16)
            et = e * t16
            oks, last = _run_masks(ids)
            for ok, idx in zip(oks, shift_idx):
                e = e + jnp.where(ok, _take16(e, idx), 0.0)
                et = et + jnp.where(ok, _take16(et, idx), 0.0)
            return ids, e, et, last

        def _sweep_sum(j, _):
            off = pl.multiple_of(j * 32, 16)
            ids_a, e_a, et_a, last_a = _seg_sums(off)
            ids_b, e_b, et_b, last_b = _seg_sums(off + 16)
            plsc.addupdate_scatter(d_tab, [ids_a], e_a, mask=last_a)
            plsc.addupdate_scatter(t_tab, [ids_a], et_a, mask=last_a)
            plsc.addupdate_scatter(d_tab, [ids_b], e_b, mask=last_b)
            plsc.addupdate_scatter(t_tab, [ids_b], et_b, mask=last_b)
            return _

        lax.fori_loop(0, nv // 2, _sweep_sum, None)

        pltpu.sync_copy(m_tab, mo_hbm.at[wid])
        pltpu.sync_copy(d_tab, do_hbm.at[wid])
        pltpu.sync_copy(t_tab, to_hbm.at[wid])

    return sc_kernel(s1, t1, segi)


def _stage3(mo_ref, do_ref, to_ref, bred_ref, wo_ref, bo_ref, out_ref):
    mw = mo_ref[...]
    m = jnp.max(mw, axis=0, keepdims=True)
    scale = jnp.where(mw == -jnp.inf, 0.0, jnp.exp(mw - m))
    d = jnp.sum(do_ref[...] * scale, axis=0, keepdims=True)
    t = jnp.sum(to_ref[...] * scale, axis=0, keepdims=True)
    wo2 = wo_ref[0:_H, :] + wo_ref[_H:2 * _H, :]
    c1 = jnp.dot(bred_ref[...], wo2, preferred_element_type=jnp.float32)
    out_ref[...] = (t / jnp.maximum(d, 1e-12)
                    + jnp.where(d > 0, 1.0, 0.0) * c1[0, 0]
                    + bo_ref[0, 0])


def kernel(annotation, segment_ids, W_reduce, b_reduce, W_gate, b_gate,
           W_out, b_out):
    segi = segment_ids.astype(jnp.int32)

    s1, t1 = pl.pallas_call(
        _stage1,
        grid=(_GRID,),
        in_specs=[
            pl.BlockSpec((_BN, _ANN), lambda i: (i, 0)),
            pl.BlockSpec((_ANN, _H), lambda i: (0, 0)),
            pl.BlockSpec((2 * _H, 1), lambda i: (0, 0)),
            pl.BlockSpec((2 * _H, 1), lambda i: (0, 0)),
        ],
        out_specs=[
            pl.BlockSpec((_BN,), lambda i: (i,)),
            pl.BlockSpec((_BN,), lambda i: (i,)),
        ],
        out_shape=[
            jax.ShapeDtypeStruct((_N,), jnp.float32),
            jax.ShapeDtypeStruct((_N,), jnp.float32),
        ],
    )(annotation, W_reduce, W_gate, W_out)

    mo, do, to = _stage2_call(s1, t1, segi)

    out = pl.pallas_call(
        _stage3,
        in_specs=[
            pl.BlockSpec((_NW, _B), lambda: (0, 0)),
            pl.BlockSpec((_NW, _B), lambda: (0, 0)),
            pl.BlockSpec((_NW, _B), lambda: (0, 0)),
            pl.BlockSpec((1, _H), lambda: (0, 0)),
            pl.BlockSpec((2 * _H, 1), lambda: (0, 0)),
            pl.BlockSpec((1, 1), lambda: (0, 0)),
        ],
        out_specs=pl.BlockSpec((1, _B), lambda: (0, 0)),
        out_shape=jax.ShapeDtypeStruct((1, _B), jnp.float32),
    )(mo, do, to, b_reduce.reshape(1, _H), W_out, b_out.reshape(1, 1))
    return out.reshape(_B)
